# baseline (speedup 1.0000x reference)
import functools
import math

import jax
import jax.numpy as jnp
from jax.experimental import pallas as pl
from jax.experimental.pallas import tpu as pltpu

N_EXP = 8
TOP_K = 2
TRAIN_CAPACITY = 1.25
MIN_CAPACITY = 4


def _capacity(num_tokens: int) -> int:
    cap = math.floor(TOP_K * TRAIN_CAPACITY * num_tokens / N_EXP)
    cap += cap % 2
    return int(max(cap, MIN_CAPACITY))


def _pass_a_kernel(cap, x_ref, wg_ref, idxs_ref, ws_ref, stats_ref,
                   c0_ref, c1_ref):
    i = pl.program_id(0)
    T = x_ref.shape[0]
    E = N_EXP

    @pl.when(i == 0)
    def _init():
        c0_ref[...] = jnp.zeros_like(c0_ref)
        c1_ref[...] = jnp.zeros_like(c1_ref)

    logits = jax.lax.dot_general(
        x_ref[...], wg_ref[...],
        dimension_numbers=(((1,), (1,)), ((), ())),
        preferred_element_type=jnp.float32,
    )

    eidx = jax.lax.broadcasted_iota(jnp.int32, (T, E), 1)
    neg_inf = jnp.float32(-jnp.inf)

    m0 = jnp.max(logits, axis=1, keepdims=True)
    e0 = jnp.min(jnp.where(logits == m0, eidx, E), axis=1, keepdims=True)
    l1 = jnp.where(eidx == e0, neg_inf, logits)
    m1 = jnp.max(l1, axis=1, keepdims=True)
    e1 = jnp.min(jnp.where(l1 == m1, eidx, E), axis=1, keepdims=True)

    z = jnp.exp(m1 - m0)
    w0 = 1.0 / (1.0 + z)
    w1 = z / (1.0 + z)

    oh0 = (eidx == e0).astype(jnp.int32)
    oh1 = (eidx == e1).astype(jnp.int32)
    ir = jax.lax.broadcasted_iota(jnp.int32, (T, T), 0)
    ic = jax.lax.broadcasted_iota(jnp.int32, (T, T), 1)
    tril = (ir >= ic).astype(jnp.float32)
    both = jnp.concatenate([oh0, oh1], axis=1).astype(jnp.float32)
    cs = jnp.dot(tril, both, preferred_element_type=jnp.float32)
    cs = cs.astype(jnp.int32)
    cs0 = cs[:, :E]
    cs1 = cs[:, E:]
    carry0 = c0_ref[...]
    carry1 = c1_ref[...]
    r0 = jnp.sum(oh0 * (carry0 + cs0), axis=1, keepdims=True) - 1
    p1 = jnp.sum(oh1 * (carry1 + cs1), axis=1, keepdims=True) - 1
    new_c0 = carry0 + cs0[T - 1:T, :]
    new_c1 = carry1 + cs1[T - 1:T, :]
    c0_ref[...] = new_c0
    c1_ref[...] = new_c1

    zeros_i = jnp.zeros((T, 1), jnp.int32)
    idxs_ref[...] = jnp.concatenate(
        [e0, e1, r0, p1, zeros_i, zeros_i, zeros_i, zeros_i], axis=1)
    zeros_f = jnp.zeros((T, 1), jnp.float32)
    ws_ref[...] = jnp.concatenate(
        [w0, w1, zeros_f, zeros_f, zeros_f, zeros_f, zeros_f, zeros_f],
        axis=1)

    stats_ref[0:1, :] = new_c0
    stats_ref[1:2, :] = jnp.minimum(new_c0 + new_c1, jnp.int32(cap))


def _pass_b_kernel(cap, idxs_ref, ws_ref, stats_ref, cb_ref, mask_ref):
    T = idxs_ref.shape[0]
    E = N_EXP

    e0 = idxs_ref[:, 0:1].reshape(T, 1, 1)
    e1 = idxs_ref[:, 1:2].reshape(T, 1, 1)
    r0 = idxs_ref[:, 2:3].reshape(T, 1, 1)
    p1 = idxs_ref[:, 3:4].reshape(T, 1, 1)
    w0 = ws_ref[:, 0:1].reshape(T, 1, 1)
    w1 = ws_ref[:, 1:2].reshape(T, 1, 1)

    tot0 = stats_ref[0:1, :]
    eidx2 = jax.lax.broadcasted_iota(jnp.int32, (T, E), 1)
    off1 = jnp.sum(jnp.where(eidx2 == idxs_ref[:, 1:2], tot0, 0),
                   axis=1, keepdims=True).reshape(T, 1, 1)
    rank1 = p1 + off1

    ie = jax.lax.broadcasted_iota(jnp.int32, (T, E, cap), 1)
    ic = jax.lax.broadcasted_iota(jnp.int32, (T, E, cap), 2)
    hit0 = (ie == e0) & (ic == r0) & (r0 < cap)
    hit1 = (ie == e1) & (ic == rank1) & (rank1 < cap)
    val = (jnp.where(hit0, w0, 0.0) + jnp.where(hit1, w1, 0.0))
    val = val.astype(jnp.float32)
    cb_ref[...] = val
    mask_ref[...] = val != 0.0


def kernel(x, W_g):
    N = x.shape[0]
    D = x.shape[2]
    E = N_EXP
    cap = _capacity(N)

    x2 = x.reshape(N, D)

    TA = 256
    nb_a = N // TA
    idxs, ws, stats = pl.pallas_call(
        functools.partial(_pass_a_kernel, cap),
        grid=(nb_a,),
        in_specs=[
            pl.BlockSpec((TA, D), lambda i: (i, 0)),
            pl.BlockSpec((E, D), lambda i: (0, 0)),
        ],
        out_specs=[
            pl.BlockSpec((TA, 8), lambda i: (i, 0)),
            pl.BlockSpec((TA, 8), lambda i: (i, 0)),
            pl.BlockSpec((2, 8), lambda i: (0, 0)),
        ],
        out_shape=[
            jax.ShapeDtypeStruct((N, 8), jnp.int32),
            jax.ShapeDtypeStruct((N, 8), jnp.float32),
            jax.ShapeDtypeStruct((2, 8), jnp.int32),
        ],
        scratch_shapes=[
            pltpu.VMEM((1, E), jnp.int32),
            pltpu.VMEM((1, E), jnp.int32),
        ],
    )(x2, W_g)

    TB = 128
    nb_b = N // TB
    cb, mask = pl.pallas_call(
        functools.partial(_pass_b_kernel, cap),
        grid=(nb_b,),
        in_specs=[
            pl.BlockSpec((TB, 8), lambda i: (i, 0)),
            pl.BlockSpec((TB, 8), lambda i: (i, 0)),
            pl.BlockSpec((2, 8), lambda i: (0, 0)),
        ],
        out_specs=[
            pl.BlockSpec((TB, E, cap), lambda i: (i, 0, 0)),
            pl.BlockSpec((TB, E, cap), lambda i: (i, 0, 0)),
        ],
        out_shape=[
            jax.ShapeDtypeStruct((N, E, cap), jnp.float32),
            jax.ShapeDtypeStruct((N, E, cap), jnp.bool_),
        ],
    )(idxs, ws, stats)

    used_capacity = stats[1, :]
    return used_capacity, cb, mask
